# baseline (speedup 1.0000x reference)
import jax
import jax.numpy as jnp
from jax.experimental import pallas as pl
from jax.experimental.pallas import tpu as pltpu

R = 2


def _body(x_ref, o_ref):
    xb = x_ref[0, :, 0]
    HH, W, Z = xb.shape
    t = jnp.swapaxes(xb, 1, 2)
    for k in range(R):
        u = t.reshape(HH, Z // R, R, W)[:, :, k, :]
        u = jnp.swapaxes(u, 1, 2)
        for j in range(R):
            o_ref[0, 0, j * R + k] = u.reshape(HH, W // R, R, Z // R)[:, :, j, :]


def kernel(x):
    B, C, H, W, Z = x.shape
    xf = x.reshape(B * C, H // R, R, W, Z)
    out = pl.pallas_call(
        _body,
        grid=(B * C, R),
        in_specs=[pl.BlockSpec((1, H // R, 1, W, Z),
                               lambda b, i: (b, 0, i, 0, 0))],
        out_specs=pl.BlockSpec((1, 1, R * R, H // R, W // R, Z // R),
                               lambda b, i: (b, i, 0, 0, 0, 0)),
        out_shape=jax.ShapeDtypeStruct(
            (B * C, R, R * R, H // R, W // R, Z // R), x.dtype),
    )(xf)
    return out.reshape(B, C * R**3, H // R, W // R, Z // R)
